# baseline (speedup 1.0000x reference)
import functools

import jax
import jax.numpy as jnp
from jax import lax
from jax.experimental import pallas as pl
from jax.experimental.pallas import tpu as pltpu
from jax.experimental.pallas import tpu_sc as plsc

NUM_ROWS = 4096 * 26
NUM_COLS = 1000
NWORDS = NUM_ROWS * NUM_COLS
NC = 2
NS = 16
NW = NC * NS
ROWS_PER_W = NUM_ROWS // NW
LANES = 16
IDXW = 128
NIDX = ROWS_PER_W // IDXW

FWORDS = 1024 * NUM_COLS
NFILL = NWORDS // FWORDS
FDEPTH = 8
assert NWORDS % FWORDS == 0

_sc_mesh = plsc.VectorSubcoreMesh(core_axis_name="c", subcore_axis_name="s")
_tc_mesh = pltpu.create_tensorcore_mesh("tc", num_cores=1)


@functools.partial(
    pl.kernel,
    out_type=(),
    mesh=_tc_mesh,
    scratch_types=(
        pltpu.SemaphoreType.DMA,
    ),
)
def _tc_zero_fill(z_hbm, out_ref, fill_sem):
    def _dma(c):
        return pltpu.make_async_copy(
            z_hbm, out_ref.at[pl.ds(c * FWORDS, FWORDS)], fill_sem)

    def _prime(c, carry):
        _dma(c).start()
        return carry

    lax.fori_loop(0, FDEPTH, _prime, 0)

    def _steady(c, carry):
        _dma(c).start()
        _dma(0).wait()
        return carry

    lax.fori_loop(FDEPTH, NFILL, _steady, 0)

    def _drain(c, carry):
        _dma(0).wait()
        return carry

    lax.fori_loop(0, FDEPTH, _drain, 0)


@functools.partial(
    pl.kernel,
    out_type=(),
    mesh=_sc_mesh,
    scratch_types=(
        pltpu.VMEM((ROWS_PER_W,), jnp.int32),
        pltpu.VMEM((NIDX, IDXW), jnp.int32),
        pltpu.VMEM((IDXW,), jnp.float32),
        pltpu.SemaphoreType.DMA,
    ),
    compiler_params=pltpu.CompilerParams(needs_layout_passes=False),
)
def _sc_scatter_ones(x_hbm, out_ref, idx_v, pos_v, ones_v, ones_sem):
    wid = lax.axis_index("s") * NC + lax.axis_index("c")
    base_row = wid * ROWS_PER_W

    pltpu.sync_copy(x_hbm.at[pl.ds(base_row, ROWS_PER_W)], idx_v)

    ones16 = jnp.ones((LANES,), jnp.float32)
    iota16 = lax.iota(jnp.int32, LANES)
    for k in range(IDXW // LANES):
        ones_v[pl.ds(k * LANES, LANES)] = ones16

    def _pos(r, carry):
        for k in range(IDXW // LANES):
            off = r * IDXW + k * LANES
            idx = idx_v[pl.ds(off, LANES)]
            pos_v[r, pl.ds(k * LANES, LANES)] = (
                (base_row + off + iota16) * NUM_COLS + idx)
        return carry

    lax.fori_loop(0, NIDX, _pos, 0)

    def _ones(r, carry):
        pltpu.make_async_copy(
            ones_v, out_ref.at[pos_v.at[r]], ones_sem).start()
        return carry

    lax.fori_loop(0, NIDX, _ones, 0)

    def _odrain(r, carry):
        pltpu.make_async_copy(
            ones_v, out_ref.at[pos_v.at[0]], ones_sem).wait()
        return carry

    lax.fori_loop(0, NIDX, _odrain, 0)


def kernel(x):
    xf = x.reshape(-1).astype(jnp.int32)
    zsrc = jnp.zeros((FWORDS,), jnp.float32)
    out_ref = jax.new_ref(pl.empty((NWORDS,), jnp.float32))
    _tc_zero_fill(zsrc, out_ref)
    _sc_scatter_ones(xf, out_ref)
    return jax.freeze(out_ref).reshape(4096, 26, NUM_COLS)


# speedup vs baseline: 11.0375x; 11.0375x over previous
import functools

import jax
import jax.numpy as jnp
from jax import lax
from jax.experimental import pallas as pl
from jax.experimental.pallas import tpu as pltpu
from jax.experimental.pallas import tpu_sc as plsc

NUM_ROWS = 4096 * 26
NUM_COLS = 1000
NWORDS = NUM_ROWS * NUM_COLS
WIN = 128
NWIN = NWORDS // WIN
NC = 2
NS = 16
NW = NC * NS
ROWS_PER_W = NUM_ROWS // NW
LANES = 16
CROWS = 128
NCHUNK = ROWS_PER_W // CROWS

FROWS = 8000
NFILL = NWIN // FROWS
FDEPTH = 8
assert NWIN % FROWS == 0

_sc_mesh = plsc.VectorSubcoreMesh(core_axis_name="c", subcore_axis_name="s")
_tc_mesh = pltpu.create_tensorcore_mesh("tc", num_cores=1)


@functools.partial(
    pl.kernel,
    out_type=(),
    mesh=_tc_mesh,
    scratch_types=(
        pltpu.VMEM((FROWS, WIN), jnp.float32),
        pltpu.SemaphoreType.DMA,
    ),
)
def _tc_zero_fill(out_ref, zbuf, fill_sem):
    zbuf[...] = jnp.zeros_like(zbuf)

    def _dma(c):
        return pltpu.make_async_copy(
            zbuf, out_ref.at[pl.ds(c * FROWS, FROWS), :], fill_sem)

    def _prime(c, carry):
        _dma(c).start()
        return carry

    lax.fori_loop(0, FDEPTH, _prime, 0)

    def _steady(c, carry):
        _dma(c).start()
        _dma(0).wait()
        return carry

    lax.fori_loop(FDEPTH, NFILL, _steady, 0)

    def _drain(c, carry):
        _dma(0).wait()
        return carry

    lax.fori_loop(0, FDEPTH, _drain, 0)


@functools.partial(
    pl.kernel,
    out_type=(),
    mesh=_sc_mesh,
    scratch_types=(
        pltpu.VMEM((ROWS_PER_W + 16,), jnp.int32),
        pltpu.VMEM((NCHUNK, WIN), jnp.int32),
        pltpu.VMEM((CROWS, WIN), jnp.float32),
        pltpu.SemaphoreType.DMA,
    ),
    compiler_params=pltpu.CompilerParams(needs_layout_passes=False),
)
def _sc_scatter_ones(x_hbm, out_ref, idxe, wv, bbuf, sem):
    wid = lax.axis_index("s") * NC + lax.axis_index("c")
    base_row = wid * ROWS_PER_W

    pltpu.sync_copy(x_hbm.at[pl.ds(base_row, ROWS_PER_W)],
                    idxe.at[pl.ds(8, ROWS_PER_W)])

    @pl.when(wid > 0)
    def _():
        pltpu.sync_copy(x_hbm.at[pl.ds(base_row - 8, 8)],
                        idxe.at[pl.ds(0, 8)])

    @pl.when(wid < NW - 1)
    def _():
        pltpu.sync_copy(x_hbm.at[pl.ds(base_row + ROWS_PER_W, 8)],
                        idxe.at[pl.ds(ROWS_PER_W + 8, 8)])

    zeros16 = jnp.zeros((LANES,), jnp.float32)
    ones16 = jnp.ones((LANES,), jnp.float32)
    iota16 = lax.iota(jnp.int32, LANES)

    def _wv(r, carry):
        for k in range(WIN // LANES):
            off = r * WIN + k * LANES
            grow = base_row + off + iota16
            p = grow * NUM_COLS + idxe[pl.ds(off + 8, LANES)]
            wv[r, pl.ds(k * LANES, LANES)] = lax.shift_right_logical(p, 7)
        return carry

    lax.fori_loop(0, NCHUNK, _wv, 0)

    def _bzero(i, carry):
        for k in range(WIN // LANES):
            bbuf[i, pl.ds(k * LANES, LANES)] = zeros16
        return carry

    lax.fori_loop(0, CROWS, _bzero, 0)

    def _paint(c, val16):
        for k in range(CROWS // LANES):
            off = c * CROWS + k * LANES
            lrow = k * LANES + iota16
            grow = base_row + off + iota16
            p = grow * NUM_COLS + idxe[pl.ds(off + 8, LANES)]
            w = lax.shift_right_logical(p, 7)
            o = lax.bitwise_and(p, WIN - 1)
            pp = (grow - 1) * NUM_COLS + idxe[pl.ds(off + 7, LANES)]
            pn = (grow + 1) * NUM_COLS + idxe[pl.ds(off + 9, LANES)]
            wp = lax.shift_right_logical(pp, 7)
            wn = lax.shift_right_logical(pn, 7)
            op = lax.bitwise_and(pp, WIN - 1)
            on = lax.bitwise_and(pn, WIN - 1)
            mp = (wp == w) & (grow > 0)
            mn = (wn == w) & (grow < NUM_ROWS - 1)
            plsc.store_scatter(bbuf, [lrow, o], val16)
            plsc.store_scatter(bbuf, [lrow, op], val16, mask=mp)
            plsc.store_scatter(bbuf, [lrow, on], val16, mask=mn)

    def _chunk(c, carry):
        _paint(c, ones16)
        cp = pltpu.make_async_copy(bbuf, out_ref.at[wv.at[c]], sem)
        cp.start()
        cp.wait()
        _paint(c, zeros16)
        return carry

    lax.fori_loop(0, NCHUNK, _chunk, 0)


def kernel(x):
    xf = x.reshape(-1).astype(jnp.int32)
    out_ref = jax.new_ref(pl.empty((NWIN, WIN), jnp.float32))
    _tc_zero_fill(out_ref)
    _sc_scatter_ones(xf, out_ref)
    return jax.freeze(out_ref).reshape(4096, 26, NUM_COLS)
